# baseline (speedup 1.0000x reference)
import functools

import jax
import jax.numpy as jnp
from jax import lax
from jax.experimental import pallas as pl
from jax.experimental.pallas import tpu as pltpu
from jax.experimental.pallas import tpu_sc as plsc

B, N, IN_DIM, DIM, K_NEI = 4, 2048, 64, 64, 16
POS_HID = 64
ATTN_HID = DIM * 4
EPS = 1e-5
M = B * N * K_NEI
_F32 = jnp.float32
_PREC = lax.Precision.DEFAULT


_TQA = 256
_NTA = N // _TQA


def _knn_body(ori_ref, pos_ref, post_ref, winT_ref, wqkvT_ref,
              q_ref, k_ref, v_ref, gidx_ref):
    b = pl.program_id(0)
    x = jnp.dot(ori_ref[0], winT_ref[...], preferred_element_type=_F32, precision=_PREC)
    qkv = jnp.dot(x, wqkvT_ref[...], preferred_element_type=_F32, precision=_PREC)
    q_ref[0] = qkv[:, 0:DIM]
    k_ref[0] = qkv[:, DIM:2 * DIM]
    v_ref[0] = qkv[:, 2 * DIM:3 * DIM]
    pq = pos_ref[0]
    pall = post_ref[0]
    d0 = pq[:, 0:1] - pall[0:1, :]
    d1 = pq[:, 1:2] - pall[1:2, :]
    d2 = pq[:, 2:3] - pall[2:3, :]
    dist = (d0 * d0 + d1 * d1) + d2 * d2
    iota = lax.broadcasted_iota(jnp.int32, (_TQA, N), 1)
    cols = []
    for _ in range(K_NEI):
        mval = jnp.min(dist, axis=1, keepdims=True)
        cand = jnp.where(dist <= mval, iota, N)
        idx = jnp.min(cand, axis=1, keepdims=True)
        cols.append(idx)
        dist = jnp.where(iota == idx, jnp.float32(1e30), dist)
    gidx_ref[0] = jnp.concatenate(cols, axis=1) + b * N


def _run_knn(ori_x, pos, post, winT, wqkvT):
    return pl.pallas_call(
        _knn_body,
        grid=(B, _NTA),
        in_specs=[
            pl.BlockSpec((1, _TQA, IN_DIM), lambda b, t: (b, t, 0)),
            pl.BlockSpec((1, _TQA, 3), lambda b, t: (b, t, 0)),
            pl.BlockSpec((1, 3, N), lambda b, t: (b, 0, 0)),
            pl.BlockSpec((IN_DIM, DIM), lambda b, t: (0, 0)),
            pl.BlockSpec((DIM, 3 * DIM), lambda b, t: (0, 0)),
        ],
        out_specs=[
            pl.BlockSpec((1, _TQA, DIM), lambda b, t: (b, t, 0)),
            pl.BlockSpec((1, _TQA, DIM), lambda b, t: (b, t, 0)),
            pl.BlockSpec((1, _TQA, DIM), lambda b, t: (b, t, 0)),
            pl.BlockSpec((1, _TQA, K_NEI), lambda b, t: (b, t, 0)),
        ],
        out_shape=[
            jax.ShapeDtypeStruct((B, N, DIM), _F32),
            jax.ShapeDtypeStruct((B, N, DIM), _F32),
            jax.ShapeDtypeStruct((B, N, DIM), _F32),
            jax.ShapeDtypeStruct((B, N, K_NEI), jnp.int32),
        ],
        compiler_params=pltpu.CompilerParams(
            dimension_semantics=("parallel", "parallel")),
    )(ori_x, pos, post, winT, wqkvT)


_SC_NC, _SC_NS = 2, 16
_NW = _SC_NC * _SC_NS
_CH = 128
_RPW = M // _NW
_NCH = _RPW // _CH


def _sc_gather(ktab, vtab, ptab, idx):
    mesh = plsc.VectorSubcoreMesh(core_axis_name="c", subcore_axis_name="s")

    @functools.partial(
        pl.kernel, mesh=mesh,
        out_type=(jax.ShapeDtypeStruct((M, DIM), _F32),
                  jax.ShapeDtypeStruct((M, DIM), _F32),
                  jax.ShapeDtypeStruct((M, 16), _F32)),
        scratch_types=[pltpu.VMEM((_CH,), jnp.int32),
                       pltpu.VMEM((_CH, DIM), _F32),
                       pltpu.VMEM((_CH, DIM), _F32),
                       pltpu.VMEM((_CH, 16), _F32),
                       pltpu.SemaphoreType.DMA],
        compiler_params=pltpu.CompilerParams(use_tc_tiling_on_sc=False),
    )
    def gk(ktab_h, vtab_h, ptab_h, idx_h, kg_h, vg_h, pg_h, idx_v, kbuf, vbuf, pbuf, sem):
        wid = lax.axis_index("s") * _SC_NC + lax.axis_index("c")

        def body(c, carry):
            base = wid * _RPW + c * _CH
            pltpu.sync_copy(idx_h.at[pl.ds(base, _CH)], idx_v)
            ck = pltpu.async_copy(ktab_h.at[idx_v], kbuf, sem)
            cv = pltpu.async_copy(vtab_h.at[idx_v], vbuf, sem)
            cp = pltpu.async_copy(ptab_h.at[idx_v], pbuf, sem)
            ck.wait()
            cv.wait()
            cp.wait()
            pltpu.sync_copy(kbuf, kg_h.at[pl.ds(base, _CH)])
            pltpu.sync_copy(vbuf, vg_h.at[pl.ds(base, _CH)])
            pltpu.sync_copy(pbuf, pg_h.at[pl.ds(base, _CH)])
            return carry

        lax.fori_loop(0, _NCH, body, 0)

    return gk(ktab, vtab, ptab, idx)


def _rep_rows(x, tq):
    c = x.shape[-1]
    return jnp.broadcast_to(x[:, None, :], (tq, K_NEI, c)).reshape(tq * K_NEI, c)


def _group_sum(x, tq):
    return jnp.sum(x.reshape(tq, K_NEI, x.shape[-1]), axis=1)


_TQ1 = 256
_TR1 = _TQ1 * K_NEI
_NT1 = (B * N) // _TQ1


def _rstat_body(pg_ref, pos_ref, sr_ref, srr_ref):
    t = pl.program_id(0)
    prep = _rep_rows(pos_ref[...], _TQ1)
    r = prep - pg_ref[...][:, 0:3]
    sr = jnp.sum(r, axis=0, keepdims=True)
    srr = lax.dot_general(r, r, (((0,), (0,)), ((), ())),
                          preferred_element_type=_F32, precision=_PREC)

    @pl.when(t == 0)
    def _():
        sr_ref[...] = jnp.zeros_like(sr_ref)
        srr_ref[...] = jnp.zeros_like(srr_ref)

    sr_ref[...] += sr
    srr_ref[...] += srr


def _run_rstat(pg, pos2d):
    return pl.pallas_call(
        _rstat_body,
        grid=(_NT1,),
        in_specs=[
            pl.BlockSpec((_TR1, 16), lambda t: (t, 0)),
            pl.BlockSpec((_TQ1, 3), lambda t: (t, 0)),
        ],
        out_specs=[
            pl.BlockSpec((1, 3), lambda t: (0, 0)),
            pl.BlockSpec((3, 3), lambda t: (0, 0)),
        ],
        out_shape=[
            jax.ShapeDtypeStruct((1, 3), _F32),
            jax.ShapeDtypeStruct((3, 3), _F32),
        ],
        compiler_params=pltpu.CompilerParams(
            dimension_semantics=("arbitrary",)),
    )(pg, pos2d)


_TQ2 = 128
_TR2 = _TQ2 * K_NEI
_NT2 = (B * N) // _TQ2


def _ustat_body(kg_ref, pg_ref, q_ref, pos_ref, w1T_ref, w2T_ref, b2_ref,
                a1_ref, c1_ref, su_ref, suu_ref):
    t = pl.program_id(0)
    prep = _rep_rows(pos_ref[...], _TQ2)
    r = prep - pg_ref[...][:, 0:3]
    h = jnp.dot(r, w1T_ref[...], preferred_element_type=_F32, precision=_PREC)
    h = jnp.maximum(h * a1_ref[...] + c1_ref[...], 0.0)
    rpe = jnp.dot(h, w2T_ref[...], preferred_element_type=_F32, precision=_PREC) + b2_ref[...]
    qrep = _rep_rows(q_ref[...], _TQ2)
    u = qrep - kg_ref[...] + rpe
    su = jnp.sum(u, axis=0, keepdims=True)
    suu = lax.dot_general(u, u, (((0,), (0,)), ((), ())),
                          preferred_element_type=_F32, precision=_PREC)

    @pl.when(t == 0)
    def _():
        su_ref[...] = jnp.zeros_like(su_ref)
        suu_ref[...] = jnp.zeros_like(suu_ref)

    su_ref[...] += su
    suu_ref[...] += suu


def _run_ustat(kg, pg, q2d, pos2d, w1T, w2T, b2, a1, c1):
    return pl.pallas_call(
        _ustat_body,
        grid=(_NT2,),
        in_specs=[
            pl.BlockSpec((_TR2, DIM), lambda t: (t, 0)),
            pl.BlockSpec((_TR2, 16), lambda t: (t, 0)),
            pl.BlockSpec((_TQ2, DIM), lambda t: (t, 0)),
            pl.BlockSpec((_TQ2, 3), lambda t: (t, 0)),
            pl.BlockSpec((3, POS_HID), lambda t: (0, 0)),
            pl.BlockSpec((POS_HID, DIM), lambda t: (0, 0)),
            pl.BlockSpec((1, DIM), lambda t: (0, 0)),
            pl.BlockSpec((1, POS_HID), lambda t: (0, 0)),
            pl.BlockSpec((1, POS_HID), lambda t: (0, 0)),
        ],
        out_specs=[
            pl.BlockSpec((1, DIM), lambda t: (0, 0)),
            pl.BlockSpec((DIM, DIM), lambda t: (0, 0)),
        ],
        out_shape=[
            jax.ShapeDtypeStruct((1, DIM), _F32),
            jax.ShapeDtypeStruct((DIM, DIM), _F32),
        ],
        compiler_params=pltpu.CompilerParams(
            dimension_semantics=("arbitrary",)),
    )(kg, pg, q2d, pos2d, w1T, w2T, b2, a1, c1)


_TQ3 = 128
_TR3 = _TQ3 * K_NEI
_NT3 = (B * N) // _TQ3
_INV_SQRT_DIM = 0.125


def _fwd_body(kg_ref, vg_ref, pg_ref, q_ref, pos_ref, ori_ref,
              w1T_ref, w2T_ref, b2_ref, a1_ref, c1_ref,
              aw1T_ref, aw2T_ref, ab2_ref, a2_ref, c2_ref, woT_ref, out_ref):
    prep = _rep_rows(pos_ref[...], _TQ3)
    r = prep - pg_ref[...][:, 0:3]
    h = jnp.dot(r, w1T_ref[...], preferred_element_type=_F32, precision=_PREC)
    h = jnp.maximum(h * a1_ref[...] + c1_ref[...], 0.0)
    rpe = jnp.dot(h, w2T_ref[...], preferred_element_type=_F32, precision=_PREC) + b2_ref[...]
    qrep = _rep_rows(q_ref[...], _TQ3)
    u = qrep - kg_ref[...] + rpe
    a = jnp.dot(u, aw1T_ref[...], preferred_element_type=_F32, precision=_PREC)
    a = jnp.maximum(a * a2_ref[...] + c2_ref[...], 0.0)
    attn = (jnp.dot(a, aw2T_ref[...], preferred_element_type=_F32, precision=_PREC)
            + ab2_ref[...]) * _INV_SQRT_DIM
    e = jnp.exp(attn)
    denom = _group_sum(e, _TQ3)
    w = vg_ref[...] + rpe
    num = _group_sum(e * w, _TQ3)
    res = num / denom
    out_ref[...] = jnp.dot(res, woT_ref[...], preferred_element_type=_F32,
                           precision=_PREC) + ori_ref[...]


def _run_fwd(kg, vg, pg, q2d, pos2d, ori2d, w1T, w2T, b2, a1, c1,
             aw1T, aw2T, ab2, a2, c2, woT):
    return pl.pallas_call(
        _fwd_body,
        grid=(_NT3,),
        in_specs=[
            pl.BlockSpec((_TR3, DIM), lambda t: (t, 0)),
            pl.BlockSpec((_TR3, DIM), lambda t: (t, 0)),
            pl.BlockSpec((_TR3, 16), lambda t: (t, 0)),
            pl.BlockSpec((_TQ3, DIM), lambda t: (t, 0)),
            pl.BlockSpec((_TQ3, 3), lambda t: (t, 0)),
            pl.BlockSpec((_TQ3, IN_DIM), lambda t: (t, 0)),
            pl.BlockSpec((3, POS_HID), lambda t: (0, 0)),
            pl.BlockSpec((POS_HID, DIM), lambda t: (0, 0)),
            pl.BlockSpec((1, DIM), lambda t: (0, 0)),
            pl.BlockSpec((1, POS_HID), lambda t: (0, 0)),
            pl.BlockSpec((1, POS_HID), lambda t: (0, 0)),
            pl.BlockSpec((DIM, ATTN_HID), lambda t: (0, 0)),
            pl.BlockSpec((ATTN_HID, DIM), lambda t: (0, 0)),
            pl.BlockSpec((1, DIM), lambda t: (0, 0)),
            pl.BlockSpec((1, ATTN_HID), lambda t: (0, 0)),
            pl.BlockSpec((1, ATTN_HID), lambda t: (0, 0)),
            pl.BlockSpec((DIM, IN_DIM), lambda t: (0, 0)),
        ],
        out_specs=pl.BlockSpec((_TQ3, IN_DIM), lambda t: (t, 0)),
        out_shape=jax.ShapeDtypeStruct((B * N, IN_DIM), _F32),
        compiler_params=pltpu.CompilerParams(
            dimension_semantics=("parallel",)),
    )(kg, vg, pg, q2d, pos2d, ori2d, w1T, w2T, b2, a1, c1,
      aw1T, aw2T, ab2, a2, c2, woT)


def kernel(ori_x, pos, W_in, W_qkv, W_out, pm_w1, pm_b1, pm_g, pm_beta, pm_w2, pm_b2,
           am_w1, am_b1, am_g, am_beta, am_w2, am_b2):
    post = jnp.swapaxes(pos, 1, 2)
    q, k, v, gidx = _run_knn(ori_x, pos, post, W_in.T, W_qkv.T)

    pos2d = pos.reshape(B * N, 3)
    ptab = jnp.pad(pos2d, ((0, 0), (0, 13)))
    kg, vg, pg = _sc_gather(k.reshape(B * N, DIM), v.reshape(B * N, DIM),
                            ptab, gidx.reshape(M))

    sr, srr = _run_rstat(pg, pos2d)
    mr = sr / M
    cov_r = srr / M - mr.T @ mr
    mean_h = mr @ pm_w1.T + pm_b1
    var_h = ((pm_w1 @ cov_r) * pm_w1).sum(axis=1)
    scale1 = (pm_g / jnp.sqrt(var_h + EPS))[None, :]
    shift1 = pm_beta[None, :] + (pm_b1[None, :] - mean_h) * scale1

    q2d = q.reshape(B * N, DIM)
    su, suu = _run_ustat(kg, pg, q2d, pos2d, pm_w1.T, pm_w2.T, pm_b2[None, :],
                         scale1, shift1)
    mu = su / M
    cov_u = suu / M - mu.T @ mu
    mean_a = mu @ am_w1.T + am_b1
    var_a = ((am_w1 @ cov_u) * am_w1).sum(axis=1)
    scale2 = (am_g / jnp.sqrt(var_a + EPS))[None, :]
    shift2 = am_beta[None, :] + (am_b1[None, :] - mean_a) * scale2

    out2d = _run_fwd(kg, vg, pg, q2d, pos2d, ori_x.reshape(B * N, IN_DIM),
                     pm_w1.T, pm_w2.T, pm_b2[None, :], scale1, shift1,
                     am_w1.T, am_w2.T, am_b2[None, :], scale2, shift2, W_out.T)
    return out2d.reshape(B, N, IN_DIM)


# speedup vs baseline: 1.0138x; 1.0138x over previous
import functools

import jax
import jax.numpy as jnp
from jax import lax
from jax.experimental import pallas as pl
from jax.experimental.pallas import tpu as pltpu
from jax.experimental.pallas import tpu_sc as plsc

B, N, IN_DIM, DIM, K_NEI = 4, 2048, 64, 64, 16
POS_HID = 64
ATTN_HID = DIM * 4
EPS = 1e-5
M = B * N * K_NEI
_F32 = jnp.float32
_PREC = lax.Precision.DEFAULT


_TQA = 256
_NTA = N // _TQA


def _knn_body(ori_ref, pos_ref, post_ref, winT_ref, wqkvT_ref,
              q_ref, k_ref, v_ref, gidx_ref, ppad_ref):
    b = pl.program_id(0)
    x = jnp.dot(ori_ref[0], winT_ref[...], preferred_element_type=_F32, precision=_PREC)
    qkv = jnp.dot(x, wqkvT_ref[...], preferred_element_type=_F32, precision=_PREC)
    q_ref[0] = qkv[:, 0:DIM]
    k_ref[0] = qkv[:, DIM:2 * DIM]
    v_ref[0] = qkv[:, 2 * DIM:3 * DIM]
    ppad_ref[0] = jnp.pad(pos_ref[0], ((0, 0), (0, 13)))
    pq = pos_ref[0]
    pall = post_ref[0]
    d0 = pq[:, 0:1] - pall[0:1, :]
    d1 = pq[:, 1:2] - pall[1:2, :]
    d2 = pq[:, 2:3] - pall[2:3, :]
    dist = (d0 * d0 + d1 * d1) + d2 * d2
    iota = lax.broadcasted_iota(jnp.int32, (_TQA, N), 1)
    cols = []
    for _ in range(K_NEI):
        mval = jnp.min(dist, axis=1, keepdims=True)
        cand = jnp.where(dist <= mval, iota, N)
        idx = jnp.min(cand, axis=1, keepdims=True)
        cols.append(idx)
        dist = jnp.where(iota == idx, jnp.float32(1e30), dist)
    gidx_ref[0] = jnp.concatenate(cols, axis=1) + b * N


def _run_knn(ori_x, pos, post, winT, wqkvT):
    return pl.pallas_call(
        _knn_body,
        grid=(B, _NTA),
        in_specs=[
            pl.BlockSpec((1, _TQA, IN_DIM), lambda b, t: (b, t, 0)),
            pl.BlockSpec((1, _TQA, 3), lambda b, t: (b, t, 0)),
            pl.BlockSpec((1, 3, N), lambda b, t: (b, 0, 0)),
            pl.BlockSpec((IN_DIM, DIM), lambda b, t: (0, 0)),
            pl.BlockSpec((DIM, 3 * DIM), lambda b, t: (0, 0)),
        ],
        out_specs=[
            pl.BlockSpec((1, _TQA, DIM), lambda b, t: (b, t, 0)),
            pl.BlockSpec((1, _TQA, DIM), lambda b, t: (b, t, 0)),
            pl.BlockSpec((1, _TQA, DIM), lambda b, t: (b, t, 0)),
            pl.BlockSpec((1, _TQA, K_NEI), lambda b, t: (b, t, 0)),
            pl.BlockSpec((1, _TQA, 16), lambda b, t: (b, t, 0)),
        ],
        out_shape=[
            jax.ShapeDtypeStruct((B, N, DIM), _F32),
            jax.ShapeDtypeStruct((B, N, DIM), _F32),
            jax.ShapeDtypeStruct((B, N, DIM), _F32),
            jax.ShapeDtypeStruct((B, N, K_NEI), jnp.int32),
            jax.ShapeDtypeStruct((B, N, 16), _F32),
        ],
        compiler_params=pltpu.CompilerParams(
            dimension_semantics=("parallel", "parallel")),
    )(ori_x, pos, post, winT, wqkvT)


_SC_NC, _SC_NS = 2, 16
_NW = _SC_NC * _SC_NS
_CH = 128
_RPW = M // _NW
_NCH = _RPW // _CH


def _sc_gather(ktab, vtab, ptab, idx):
    mesh = plsc.VectorSubcoreMesh(core_axis_name="c", subcore_axis_name="s")

    @functools.partial(
        pl.kernel, mesh=mesh,
        out_type=(jax.ShapeDtypeStruct((M, DIM), _F32),
                  jax.ShapeDtypeStruct((M, DIM), _F32),
                  jax.ShapeDtypeStruct((M, 16), _F32)),
        scratch_types=[pltpu.VMEM((_CH,), jnp.int32),
                       pltpu.VMEM((_CH, DIM), _F32),
                       pltpu.VMEM((_CH, DIM), _F32),
                       pltpu.VMEM((_CH, 16), _F32),
                       pltpu.SemaphoreType.DMA],
        compiler_params=pltpu.CompilerParams(use_tc_tiling_on_sc=False),
    )
    def gk(ktab_h, vtab_h, ptab_h, idx_h, kg_h, vg_h, pg_h, idx_v, kbuf, vbuf, pbuf, sem):
        wid = lax.axis_index("s") * _SC_NC + lax.axis_index("c")

        def body(c, carry):
            base = wid * _RPW + c * _CH
            pltpu.sync_copy(idx_h.at[pl.ds(base, _CH)], idx_v)
            ck = pltpu.async_copy(ktab_h.at[idx_v], kbuf, sem)
            cv = pltpu.async_copy(vtab_h.at[idx_v], vbuf, sem)
            cp = pltpu.async_copy(ptab_h.at[idx_v], pbuf, sem)
            ck.wait()
            cv.wait()
            cp.wait()
            pltpu.sync_copy(kbuf, kg_h.at[pl.ds(base, _CH)])
            pltpu.sync_copy(vbuf, vg_h.at[pl.ds(base, _CH)])
            pltpu.sync_copy(pbuf, pg_h.at[pl.ds(base, _CH)])
            return carry

        lax.fori_loop(0, _NCH, body, 0)

    return gk(ktab, vtab, ptab, idx)


def _rep_rows(x, tq):
    c = x.shape[-1]
    return jnp.broadcast_to(x[:, None, :], (tq, K_NEI, c)).reshape(tq * K_NEI, c)


def _group_sum(x, tq):
    return jnp.sum(x.reshape(tq, K_NEI, x.shape[-1]), axis=1)


_TQ1 = 256
_TR1 = _TQ1 * K_NEI
_NT1 = (B * N) // _TQ1


def _rstat_body(pg_ref, pos_ref, sr_ref, srr_ref):
    t = pl.program_id(0)
    prep = _rep_rows(pos_ref[...], _TQ1)
    r = prep - pg_ref[...][:, 0:3]
    sr = jnp.sum(r, axis=0, keepdims=True)
    srr = lax.dot_general(r, r, (((0,), (0,)), ((), ())),
                          preferred_element_type=_F32, precision=_PREC)

    @pl.when(t == 0)
    def _():
        sr_ref[...] = jnp.zeros_like(sr_ref)
        srr_ref[...] = jnp.zeros_like(srr_ref)

    sr_ref[...] += sr
    srr_ref[...] += srr


def _run_rstat(pg, pos2d):
    return pl.pallas_call(
        _rstat_body,
        grid=(_NT1,),
        in_specs=[
            pl.BlockSpec((_TR1, 16), lambda t: (t, 0)),
            pl.BlockSpec((_TQ1, 3), lambda t: (t, 0)),
        ],
        out_specs=[
            pl.BlockSpec((1, 3), lambda t: (0, 0)),
            pl.BlockSpec((3, 3), lambda t: (0, 0)),
        ],
        out_shape=[
            jax.ShapeDtypeStruct((1, 3), _F32),
            jax.ShapeDtypeStruct((3, 3), _F32),
        ],
        compiler_params=pltpu.CompilerParams(
            dimension_semantics=("arbitrary",)),
    )(pg, pos2d)


_TQ2 = 128
_TR2 = _TQ2 * K_NEI
_NT2 = (B * N) // _TQ2


def _ustat_body(kg_ref, pg_ref, q_ref, pos_ref, w1T_ref, w2T_ref, b2_ref,
                a1_ref, c1_ref, su_ref, suu_ref):
    t = pl.program_id(0)
    prep = _rep_rows(pos_ref[...], _TQ2)
    r = prep - pg_ref[...][:, 0:3]
    h = jnp.dot(r, w1T_ref[...], preferred_element_type=_F32, precision=_PREC)
    h = jnp.maximum(h * a1_ref[...] + c1_ref[...], 0.0)
    rpe = jnp.dot(h, w2T_ref[...], preferred_element_type=_F32, precision=_PREC) + b2_ref[...]
    qrep = _rep_rows(q_ref[...], _TQ2)
    u = qrep - kg_ref[...] + rpe
    su = jnp.sum(u, axis=0, keepdims=True)
    suu = lax.dot_general(u, u, (((0,), (0,)), ((), ())),
                          preferred_element_type=_F32, precision=_PREC)

    @pl.when(t == 0)
    def _():
        su_ref[...] = jnp.zeros_like(su_ref)
        suu_ref[...] = jnp.zeros_like(suu_ref)

    su_ref[...] += su
    suu_ref[...] += suu


def _run_ustat(kg, pg, q2d, pos2d, w1T, w2T, b2, a1, c1):
    return pl.pallas_call(
        _ustat_body,
        grid=(_NT2,),
        in_specs=[
            pl.BlockSpec((_TR2, DIM), lambda t: (t, 0)),
            pl.BlockSpec((_TR2, 16), lambda t: (t, 0)),
            pl.BlockSpec((_TQ2, DIM), lambda t: (t, 0)),
            pl.BlockSpec((_TQ2, 3), lambda t: (t, 0)),
            pl.BlockSpec((3, POS_HID), lambda t: (0, 0)),
            pl.BlockSpec((POS_HID, DIM), lambda t: (0, 0)),
            pl.BlockSpec((1, DIM), lambda t: (0, 0)),
            pl.BlockSpec((1, POS_HID), lambda t: (0, 0)),
            pl.BlockSpec((1, POS_HID), lambda t: (0, 0)),
        ],
        out_specs=[
            pl.BlockSpec((1, DIM), lambda t: (0, 0)),
            pl.BlockSpec((DIM, DIM), lambda t: (0, 0)),
        ],
        out_shape=[
            jax.ShapeDtypeStruct((1, DIM), _F32),
            jax.ShapeDtypeStruct((DIM, DIM), _F32),
        ],
        compiler_params=pltpu.CompilerParams(
            dimension_semantics=("arbitrary",)),
    )(kg, pg, q2d, pos2d, w1T, w2T, b2, a1, c1)


def _fin_body(s_ref, ss_ref, w1T_ref, b1_ref, g_ref, beta_ref, scale_ref, shift_ref):
    m = s_ref[...] * (1.0 / M)
    cov = ss_ref[...] * (1.0 / M) - lax.dot_general(
        m, m, (((0,), (0,)), ((), ())), preferred_element_type=_F32, precision=_PREC)
    tmp = jnp.dot(cov, w1T_ref[...], preferred_element_type=_F32, precision=_PREC)
    var = jnp.sum(tmp * w1T_ref[...], axis=0, keepdims=True)
    mean = jnp.dot(m, w1T_ref[...], preferred_element_type=_F32,
                   precision=_PREC) + b1_ref[...]
    scale = g_ref[...] * lax.rsqrt(var + EPS)
    scale_ref[...] = scale
    shift_ref[...] = beta_ref[...] + (b1_ref[...] - mean) * scale


def _run_fin(s, ss, w1T, b1, g, beta):
    din, dout = w1T.shape
    full = lambda shape: pl.BlockSpec(shape, lambda: tuple(0 for _ in shape))
    return pl.pallas_call(
        _fin_body,
        in_specs=[full((1, din)), full((din, din)), full((din, dout)),
                  full((1, dout)), full((1, dout)), full((1, dout))],
        out_specs=[full((1, dout)), full((1, dout))],
        out_shape=[jax.ShapeDtypeStruct((1, dout), _F32),
                   jax.ShapeDtypeStruct((1, dout), _F32)],
    )(s, ss, w1T, b1, g, beta)


_TQ3 = 128
_TR3 = _TQ3 * K_NEI
_NT3 = (B * N) // _TQ3
_INV_SQRT_DIM = 0.125


def _fwd_body(kg_ref, vg_ref, pg_ref, q_ref, pos_ref, ori_ref,
              w1T_ref, w2T_ref, b2_ref, a1_ref, c1_ref,
              aw1T_ref, aw2T_ref, ab2_ref, a2_ref, c2_ref, woT_ref, out_ref):
    prep = _rep_rows(pos_ref[...], _TQ3)
    r = prep - pg_ref[...][:, 0:3]
    h = jnp.dot(r, w1T_ref[...], preferred_element_type=_F32, precision=_PREC)
    h = jnp.maximum(h * a1_ref[...] + c1_ref[...], 0.0)
    rpe = jnp.dot(h, w2T_ref[...], preferred_element_type=_F32, precision=_PREC) + b2_ref[...]
    qrep = _rep_rows(q_ref[...], _TQ3)
    u = qrep - kg_ref[...] + rpe
    a = jnp.dot(u, aw1T_ref[...], preferred_element_type=_F32, precision=_PREC)
    a = jnp.maximum(a * a2_ref[...] + c2_ref[...], 0.0)
    attn = (jnp.dot(a, aw2T_ref[...], preferred_element_type=_F32, precision=_PREC)
            + ab2_ref[...]) * _INV_SQRT_DIM
    e = jnp.exp(attn)
    denom = _group_sum(e, _TQ3)
    w = vg_ref[...] + rpe
    num = _group_sum(e * w, _TQ3)
    res = num / denom
    out_ref[...] = jnp.dot(res, woT_ref[...], preferred_element_type=_F32,
                           precision=_PREC) + ori_ref[...]


def _run_fwd(kg, vg, pg, q2d, pos2d, ori2d, w1T, w2T, b2, a1, c1,
             aw1T, aw2T, ab2, a2, c2, woT):
    return pl.pallas_call(
        _fwd_body,
        grid=(_NT3,),
        in_specs=[
            pl.BlockSpec((_TR3, DIM), lambda t: (t, 0)),
            pl.BlockSpec((_TR3, DIM), lambda t: (t, 0)),
            pl.BlockSpec((_TR3, 16), lambda t: (t, 0)),
            pl.BlockSpec((_TQ3, DIM), lambda t: (t, 0)),
            pl.BlockSpec((_TQ3, 3), lambda t: (t, 0)),
            pl.BlockSpec((_TQ3, IN_DIM), lambda t: (t, 0)),
            pl.BlockSpec((3, POS_HID), lambda t: (0, 0)),
            pl.BlockSpec((POS_HID, DIM), lambda t: (0, 0)),
            pl.BlockSpec((1, DIM), lambda t: (0, 0)),
            pl.BlockSpec((1, POS_HID), lambda t: (0, 0)),
            pl.BlockSpec((1, POS_HID), lambda t: (0, 0)),
            pl.BlockSpec((DIM, ATTN_HID), lambda t: (0, 0)),
            pl.BlockSpec((ATTN_HID, DIM), lambda t: (0, 0)),
            pl.BlockSpec((1, DIM), lambda t: (0, 0)),
            pl.BlockSpec((1, ATTN_HID), lambda t: (0, 0)),
            pl.BlockSpec((1, ATTN_HID), lambda t: (0, 0)),
            pl.BlockSpec((DIM, IN_DIM), lambda t: (0, 0)),
        ],
        out_specs=pl.BlockSpec((_TQ3, IN_DIM), lambda t: (t, 0)),
        out_shape=jax.ShapeDtypeStruct((B * N, IN_DIM), _F32),
        compiler_params=pltpu.CompilerParams(
            dimension_semantics=("parallel",)),
    )(kg, vg, pg, q2d, pos2d, ori2d, w1T, w2T, b2, a1, c1,
      aw1T, aw2T, ab2, a2, c2, woT)


def kernel(ori_x, pos, W_in, W_qkv, W_out, pm_w1, pm_b1, pm_g, pm_beta, pm_w2, pm_b2,
           am_w1, am_b1, am_g, am_beta, am_w2, am_b2):
    post = jnp.swapaxes(pos, 1, 2)
    q, k, v, gidx, ppad = _run_knn(ori_x, pos, post, W_in.T, W_qkv.T)

    pos2d = pos.reshape(B * N, 3)
    kg, vg, pg = _sc_gather(k.reshape(B * N, DIM), v.reshape(B * N, DIM),
                            ppad.reshape(B * N, 16), gidx.reshape(M))

    sr, srr = _run_rstat(pg, pos2d)
    scale1, shift1 = _run_fin(sr, srr, pm_w1.T, pm_b1[None, :], pm_g[None, :],
                              pm_beta[None, :])

    q2d = q.reshape(B * N, DIM)
    su, suu = _run_ustat(kg, pg, q2d, pos2d, pm_w1.T, pm_w2.T, pm_b2[None, :],
                         scale1, shift1)
    scale2, shift2 = _run_fin(su, suu, am_w1.T, am_b1[None, :], am_g[None, :],
                              am_beta[None, :])

    out2d = _run_fwd(kg, vg, pg, q2d, pos2d, ori_x.reshape(B * N, IN_DIM),
                     pm_w1.T, pm_w2.T, pm_b2[None, :], scale1, shift1,
                     am_w1.T, am_w2.T, am_b2[None, :], scale2, shift2, W_out.T)
    return out2d.reshape(B, N, IN_DIM)


# speedup vs baseline: 1.0647x; 1.0502x over previous
import functools

import jax
import jax.numpy as jnp
from jax import lax
from jax.experimental import pallas as pl
from jax.experimental.pallas import tpu as pltpu
from jax.experimental.pallas import tpu_sc as plsc

B, N, IN_DIM, DIM, K_NEI = 4, 2048, 64, 64, 16
_TW = 2 * DIM + 16
POS_HID = 64
ATTN_HID = DIM * 4
EPS = 1e-5
M = B * N * K_NEI
_F32 = jnp.float32
_PREC = lax.Precision.DEFAULT


_TQA = 256
_NTA = N // _TQA


def _knn_body(ori_ref, pos_ref, post_ref, winT_ref, wqkvT_ref,
              q_ref, kvp_ref, gidx_ref):
    b = pl.program_id(0)
    x = jnp.dot(ori_ref[0], winT_ref[...], preferred_element_type=_F32, precision=_PREC)
    qkv = jnp.dot(x, wqkvT_ref[...], preferred_element_type=_F32, precision=_PREC)
    q_ref[0] = qkv[:, 0:DIM]
    kvp_ref[0] = jnp.concatenate(
        [qkv[:, DIM:3 * DIM], jnp.pad(pos_ref[0], ((0, 0), (0, 13)))], axis=1)
    pq = pos_ref[0]
    pall = post_ref[0]
    d0 = pq[:, 0:1] - pall[0:1, :]
    d1 = pq[:, 1:2] - pall[1:2, :]
    d2 = pq[:, 2:3] - pall[2:3, :]
    dist = (d0 * d0 + d1 * d1) + d2 * d2
    iota = lax.broadcasted_iota(jnp.int32, (_TQA, N), 1)
    cols = []
    for _ in range(K_NEI):
        mval = jnp.min(dist, axis=1, keepdims=True)
        cand = jnp.where(dist <= mval, iota, N)
        idx = jnp.min(cand, axis=1, keepdims=True)
        cols.append(idx)
        dist = jnp.where(iota == idx, jnp.float32(1e30), dist)
    gidx_ref[0] = jnp.concatenate(cols, axis=1) + b * N


def _run_knn(ori_x, pos, post, winT, wqkvT):
    return pl.pallas_call(
        _knn_body,
        grid=(B, _NTA),
        in_specs=[
            pl.BlockSpec((1, _TQA, IN_DIM), lambda b, t: (b, t, 0)),
            pl.BlockSpec((1, _TQA, 3), lambda b, t: (b, t, 0)),
            pl.BlockSpec((1, 3, N), lambda b, t: (b, 0, 0)),
            pl.BlockSpec((IN_DIM, DIM), lambda b, t: (0, 0)),
            pl.BlockSpec((DIM, 3 * DIM), lambda b, t: (0, 0)),
        ],
        out_specs=[
            pl.BlockSpec((1, _TQA, DIM), lambda b, t: (b, t, 0)),
            pl.BlockSpec((1, _TQA, _TW), lambda b, t: (b, t, 0)),
            pl.BlockSpec((1, _TQA, K_NEI), lambda b, t: (b, t, 0)),
        ],
        out_shape=[
            jax.ShapeDtypeStruct((B, N, DIM), _F32),
            jax.ShapeDtypeStruct((B, N, _TW), _F32),
            jax.ShapeDtypeStruct((B, N, K_NEI), jnp.int32),
        ],
        compiler_params=pltpu.CompilerParams(
            dimension_semantics=("parallel", "parallel")),
    )(ori_x, pos, post, winT, wqkvT)


_SC_NC, _SC_NS = 2, 16
_NW = _SC_NC * _SC_NS
_CH = 128
_RPW = M // _NW
_NCH = _RPW // _CH


def _sc_gather(tab, idx):
    mesh = plsc.VectorSubcoreMesh(core_axis_name="c", subcore_axis_name="s")

    @functools.partial(
        pl.kernel, mesh=mesh,
        out_type=jax.ShapeDtypeStruct((M, _TW), _F32),
        scratch_types=[pltpu.VMEM((_CH,), jnp.int32),
                       pltpu.VMEM((_CH,), jnp.int32),
                       pltpu.VMEM((_CH, _TW), _F32),
                       pltpu.VMEM((_CH, _TW), _F32),
                       pltpu.SemaphoreType.DMA,
                       pltpu.SemaphoreType.DMA],
        compiler_params=pltpu.CompilerParams(use_tc_tiling_on_sc=False),
    )
    def gk(tab_h, idx_h, gt_h, idx0, idx1, buf0, buf1, sem0, sem1):
        wid = lax.axis_index("s") * _SC_NC + lax.axis_index("c")
        idxs, bufs, sems = (idx0, idx1), (buf0, buf1), (sem0, sem1)
        gops = [None, None]
        for c in range(_NCH):
            p = c & 1
            base = wid * _RPW + c * _CH
            pltpu.sync_copy(idx_h.at[pl.ds(base, _CH)], idxs[p])
            gops[p] = pltpu.async_copy(tab_h.at[idxs[p]], bufs[p], sems[p])
            if c >= 1:
                gops[1 - p].wait()
                pltpu.sync_copy(bufs[1 - p],
                                gt_h.at[pl.ds(base - _CH, _CH)])
        p = (_NCH - 1) & 1
        gops[p].wait()
        pltpu.sync_copy(bufs[p], gt_h.at[pl.ds(wid * _RPW + (_NCH - 1) * _CH, _CH)])

    return gk(tab, idx)


def _rep_rows(x, tq):
    c = x.shape[-1]
    return jnp.broadcast_to(x[:, None, :], (tq, K_NEI, c)).reshape(tq * K_NEI, c)


def _group_sum(x, tq):
    return jnp.sum(x.reshape(tq, K_NEI, x.shape[-1]), axis=1)


_TQ1 = 256
_TR1 = _TQ1 * K_NEI
_NT1 = (B * N) // _TQ1


def _rstat_body(gt_ref, pos_ref, sr_ref, srr_ref):
    t = pl.program_id(0)
    prep = _rep_rows(pos_ref[...], _TQ1)
    r = prep - gt_ref[...][:, 2 * DIM:2 * DIM + 3]
    sr = jnp.sum(r, axis=0, keepdims=True)
    srr = lax.dot_general(r, r, (((0,), (0,)), ((), ())),
                          preferred_element_type=_F32, precision=_PREC)

    @pl.when(t == 0)
    def _():
        sr_ref[...] = jnp.zeros_like(sr_ref)
        srr_ref[...] = jnp.zeros_like(srr_ref)

    sr_ref[...] += sr
    srr_ref[...] += srr


def _run_rstat(gt, pos2d):
    return pl.pallas_call(
        _rstat_body,
        grid=(_NT1,),
        in_specs=[
            pl.BlockSpec((_TR1, _TW), lambda t: (t, 0)),
            pl.BlockSpec((_TQ1, 3), lambda t: (t, 0)),
        ],
        out_specs=[
            pl.BlockSpec((1, 3), lambda t: (0, 0)),
            pl.BlockSpec((3, 3), lambda t: (0, 0)),
        ],
        out_shape=[
            jax.ShapeDtypeStruct((1, 3), _F32),
            jax.ShapeDtypeStruct((3, 3), _F32),
        ],
        compiler_params=pltpu.CompilerParams(
            dimension_semantics=("arbitrary",)),
    )(gt, pos2d)


_TQ2 = 128
_TR2 = _TQ2 * K_NEI
_NT2 = (B * N) // _TQ2


def _ustat_body(gt_ref, q_ref, pos_ref, w1T_ref, w2T_ref, b2_ref,
                a1_ref, c1_ref, su_ref, suu_ref):
    t = pl.program_id(0)
    gt = gt_ref[...]
    prep = _rep_rows(pos_ref[...], _TQ2)
    r = prep - gt[:, 2 * DIM:2 * DIM + 3]
    h = jnp.dot(r, w1T_ref[...], preferred_element_type=_F32, precision=_PREC)
    h = jnp.maximum(h * a1_ref[...] + c1_ref[...], 0.0)
    rpe = jnp.dot(h, w2T_ref[...], preferred_element_type=_F32, precision=_PREC) + b2_ref[...]
    qrep = _rep_rows(q_ref[...], _TQ2)
    u = qrep - gt[:, 0:DIM] + rpe
    su = jnp.sum(u, axis=0, keepdims=True)
    suu = lax.dot_general(u, u, (((0,), (0,)), ((), ())),
                          preferred_element_type=_F32, precision=_PREC)

    @pl.when(t == 0)
    def _():
        su_ref[...] = jnp.zeros_like(su_ref)
        suu_ref[...] = jnp.zeros_like(suu_ref)

    su_ref[...] += su
    suu_ref[...] += suu


def _run_ustat(gt, q2d, pos2d, w1T, w2T, b2, a1, c1):
    return pl.pallas_call(
        _ustat_body,
        grid=(_NT2,),
        in_specs=[
            pl.BlockSpec((_TR2, _TW), lambda t: (t, 0)),
            pl.BlockSpec((_TQ2, DIM), lambda t: (t, 0)),
            pl.BlockSpec((_TQ2, 3), lambda t: (t, 0)),
            pl.BlockSpec((3, POS_HID), lambda t: (0, 0)),
            pl.BlockSpec((POS_HID, DIM), lambda t: (0, 0)),
            pl.BlockSpec((1, DIM), lambda t: (0, 0)),
            pl.BlockSpec((1, POS_HID), lambda t: (0, 0)),
            pl.BlockSpec((1, POS_HID), lambda t: (0, 0)),
        ],
        out_specs=[
            pl.BlockSpec((1, DIM), lambda t: (0, 0)),
            pl.BlockSpec((DIM, DIM), lambda t: (0, 0)),
        ],
        out_shape=[
            jax.ShapeDtypeStruct((1, DIM), _F32),
            jax.ShapeDtypeStruct((DIM, DIM), _F32),
        ],
        compiler_params=pltpu.CompilerParams(
            dimension_semantics=("arbitrary",)),
    )(gt, q2d, pos2d, w1T, w2T, b2, a1, c1)


def _fin_body(s_ref, ss_ref, w1T_ref, b1_ref, g_ref, beta_ref, scale_ref, shift_ref):
    m = s_ref[...] * (1.0 / M)
    cov = ss_ref[...] * (1.0 / M) - lax.dot_general(
        m, m, (((0,), (0,)), ((), ())), preferred_element_type=_F32, precision=_PREC)
    tmp = jnp.dot(cov, w1T_ref[...], preferred_element_type=_F32, precision=_PREC)
    var = jnp.sum(tmp * w1T_ref[...], axis=0, keepdims=True)
    mean = jnp.dot(m, w1T_ref[...], preferred_element_type=_F32,
                   precision=_PREC) + b1_ref[...]
    scale = g_ref[...] * lax.rsqrt(var + EPS)
    scale_ref[...] = scale
    shift_ref[...] = beta_ref[...] + (b1_ref[...] - mean) * scale


def _run_fin(s, ss, w1T, b1, g, beta):
    din, dout = w1T.shape
    full = lambda shape: pl.BlockSpec(shape, lambda: tuple(0 for _ in shape))
    return pl.pallas_call(
        _fin_body,
        in_specs=[full((1, din)), full((din, din)), full((din, dout)),
                  full((1, dout)), full((1, dout)), full((1, dout))],
        out_specs=[full((1, dout)), full((1, dout))],
        out_shape=[jax.ShapeDtypeStruct((1, dout), _F32),
                   jax.ShapeDtypeStruct((1, dout), _F32)],
    )(s, ss, w1T, b1, g, beta)


_TQ3 = 128
_TR3 = _TQ3 * K_NEI
_NT3 = (B * N) // _TQ3
_INV_SQRT_DIM = 0.125


def _fwd_body(gt_ref, q_ref, pos_ref, ori_ref,
              w1T_ref, w2T_ref, b2_ref, a1_ref, c1_ref,
              aw1T_ref, aw2T_ref, ab2_ref, a2_ref, c2_ref, woT_ref, out_ref):
    gt = gt_ref[...]
    prep = _rep_rows(pos_ref[...], _TQ3)
    r = prep - gt[:, 2 * DIM:2 * DIM + 3]
    h = jnp.dot(r, w1T_ref[...], preferred_element_type=_F32, precision=_PREC)
    h = jnp.maximum(h * a1_ref[...] + c1_ref[...], 0.0)
    rpe = jnp.dot(h, w2T_ref[...], preferred_element_type=_F32, precision=_PREC) + b2_ref[...]
    qrep = _rep_rows(q_ref[...], _TQ3)
    u = qrep - gt[:, 0:DIM] + rpe
    a = jnp.dot(u, aw1T_ref[...], preferred_element_type=_F32, precision=_PREC)
    a = jnp.maximum(a * a2_ref[...] + c2_ref[...], 0.0)
    attn = (jnp.dot(a, aw2T_ref[...], preferred_element_type=_F32, precision=_PREC)
            + ab2_ref[...]) * _INV_SQRT_DIM
    e = jnp.exp(attn)
    denom = _group_sum(e, _TQ3)
    w = gt[:, DIM:2 * DIM] + rpe
    num = _group_sum(e * w, _TQ3)
    res = num / denom
    out_ref[...] = jnp.dot(res, woT_ref[...], preferred_element_type=_F32,
                           precision=_PREC) + ori_ref[...]


def _run_fwd(gt, q2d, pos2d, ori2d, w1T, w2T, b2, a1, c1,
             aw1T, aw2T, ab2, a2, c2, woT):
    return pl.pallas_call(
        _fwd_body,
        grid=(_NT3,),
        in_specs=[
            pl.BlockSpec((_TR3, _TW), lambda t: (t, 0)),
            pl.BlockSpec((_TQ3, DIM), lambda t: (t, 0)),
            pl.BlockSpec((_TQ3, 3), lambda t: (t, 0)),
            pl.BlockSpec((_TQ3, IN_DIM), lambda t: (t, 0)),
            pl.BlockSpec((3, POS_HID), lambda t: (0, 0)),
            pl.BlockSpec((POS_HID, DIM), lambda t: (0, 0)),
            pl.BlockSpec((1, DIM), lambda t: (0, 0)),
            pl.BlockSpec((1, POS_HID), lambda t: (0, 0)),
            pl.BlockSpec((1, POS_HID), lambda t: (0, 0)),
            pl.BlockSpec((DIM, ATTN_HID), lambda t: (0, 0)),
            pl.BlockSpec((ATTN_HID, DIM), lambda t: (0, 0)),
            pl.BlockSpec((1, DIM), lambda t: (0, 0)),
            pl.BlockSpec((1, ATTN_HID), lambda t: (0, 0)),
            pl.BlockSpec((1, ATTN_HID), lambda t: (0, 0)),
            pl.BlockSpec((DIM, IN_DIM), lambda t: (0, 0)),
        ],
        out_specs=pl.BlockSpec((_TQ3, IN_DIM), lambda t: (t, 0)),
        out_shape=jax.ShapeDtypeStruct((B * N, IN_DIM), _F32),
        compiler_params=pltpu.CompilerParams(
            dimension_semantics=("parallel",)),
    )(gt, q2d, pos2d, ori2d, w1T, w2T, b2, a1, c1,
      aw1T, aw2T, ab2, a2, c2, woT)


def kernel(ori_x, pos, W_in, W_qkv, W_out, pm_w1, pm_b1, pm_g, pm_beta, pm_w2, pm_b2,
           am_w1, am_b1, am_g, am_beta, am_w2, am_b2):
    post = jnp.swapaxes(pos, 1, 2)
    q, kvp, gidx = _run_knn(ori_x, pos, post, W_in.T, W_qkv.T)

    pos2d = pos.reshape(B * N, 3)
    gt = _sc_gather(kvp.reshape(B * N, _TW), gidx.reshape(M))

    sr, srr = _run_rstat(gt, pos2d)
    scale1, shift1 = _run_fin(sr, srr, pm_w1.T, pm_b1[None, :], pm_g[None, :],
                              pm_beta[None, :])

    q2d = q.reshape(B * N, DIM)
    su, suu = _run_ustat(gt, q2d, pos2d, pm_w1.T, pm_w2.T, pm_b2[None, :],
                         scale1, shift1)
    scale2, shift2 = _run_fin(su, suu, am_w1.T, am_b1[None, :], am_g[None, :],
                              am_beta[None, :])

    out2d = _run_fwd(gt, q2d, pos2d, ori_x.reshape(B * N, IN_DIM),
                     pm_w1.T, pm_w2.T, pm_b2[None, :], scale1, shift1,
                     am_w1.T, am_w2.T, am_b2[None, :], scale2, shift2, W_out.T)
    return out2d.reshape(B, N, IN_DIM)


# speedup vs baseline: 1.0730x; 1.0078x over previous
import functools

import jax
import jax.numpy as jnp
from jax import lax
from jax.experimental import pallas as pl
from jax.experimental.pallas import tpu as pltpu
from jax.experimental.pallas import tpu_sc as plsc

B, N, IN_DIM, DIM, K_NEI = 4, 2048, 64, 64, 16
_TW = 2 * DIM + 16
POS_HID = 64
ATTN_HID = DIM * 4
EPS = 1e-5
M = B * N * K_NEI
_F32 = jnp.float32
_PREC = lax.Precision.DEFAULT


_TQA = 256
_NTA = N // _TQA


def _knn_body(ori_ref, pos_ref, post_ref, winT_ref, wqkvT_ref,
              q_ref, kvp_ref, gidx_ref):
    b = pl.program_id(0)
    x = jnp.dot(ori_ref[0], winT_ref[...], preferred_element_type=_F32, precision=_PREC)
    qkv = jnp.dot(x, wqkvT_ref[...], preferred_element_type=_F32, precision=_PREC)
    q_ref[0] = qkv[:, 0:DIM]
    kvp_ref[0] = jnp.concatenate(
        [qkv[:, DIM:3 * DIM], jnp.pad(pos_ref[0], ((0, 0), (0, 13)))], axis=1)
    pq = pos_ref[0]
    pall = post_ref[0]
    d0 = pq[:, 0:1] - pall[0:1, :]
    d1 = pq[:, 1:2] - pall[1:2, :]
    d2 = pq[:, 2:3] - pall[2:3, :]
    dist = (d0 * d0 + d1 * d1) + d2 * d2
    iota = lax.broadcasted_iota(jnp.int32, (_TQA, N), 1)
    cols = []
    for _ in range(K_NEI):
        mval = jnp.min(dist, axis=1, keepdims=True)
        cand = jnp.where(dist <= mval, iota, N)
        idx = jnp.min(cand, axis=1, keepdims=True)
        cols.append(idx)
        dist = jnp.where(iota == idx, jnp.float32(1e30), dist)
    gidx_ref[0] = jnp.concatenate(cols, axis=1) + b * N


def _run_knn(ori_x, pos, post, winT, wqkvT):
    return pl.pallas_call(
        _knn_body,
        grid=(B, _NTA),
        in_specs=[
            pl.BlockSpec((1, _TQA, IN_DIM), lambda b, t: (b, t, 0)),
            pl.BlockSpec((1, _TQA, 3), lambda b, t: (b, t, 0)),
            pl.BlockSpec((1, 3, N), lambda b, t: (b, 0, 0)),
            pl.BlockSpec((IN_DIM, DIM), lambda b, t: (0, 0)),
            pl.BlockSpec((DIM, 3 * DIM), lambda b, t: (0, 0)),
        ],
        out_specs=[
            pl.BlockSpec((1, _TQA, DIM), lambda b, t: (b, t, 0)),
            pl.BlockSpec((1, _TQA, _TW), lambda b, t: (b, t, 0)),
            pl.BlockSpec((1, _TQA, K_NEI), lambda b, t: (b, t, 0)),
        ],
        out_shape=[
            jax.ShapeDtypeStruct((B, N, DIM), _F32),
            jax.ShapeDtypeStruct((B, N, _TW), _F32),
            jax.ShapeDtypeStruct((B, N, K_NEI), jnp.int32),
        ],
        compiler_params=pltpu.CompilerParams(
            dimension_semantics=("parallel", "parallel")),
    )(ori_x, pos, post, winT, wqkvT)


_SC_NC, _SC_NS = 2, 16
_NW = _SC_NC * _SC_NS
_CH = 128
_RPW = M // _NW
_NCH = _RPW // _CH


def _sc_gather(tab, idx):
    mesh = plsc.VectorSubcoreMesh(core_axis_name="c", subcore_axis_name="s")

    _DEPTH = 4

    @functools.partial(
        pl.kernel, mesh=mesh,
        out_type=jax.ShapeDtypeStruct((M, _TW), _F32),
        scratch_types=[pltpu.VMEM((_RPW,), jnp.int32)]
                      + [pltpu.VMEM((_CH, _TW), _F32)] * _DEPTH
                      + [pltpu.SemaphoreType.DMA] * _DEPTH,
        compiler_params=pltpu.CompilerParams(use_tc_tiling_on_sc=False),
    )
    def gk(tab_h, idx_h, gt_h, idxv, b0, b1, b2, b3, s0, s1, s2, s3):
        wid = lax.axis_index("s") * _SC_NC + lax.axis_index("c")
        base = wid * _RPW
        pltpu.sync_copy(idx_h.at[pl.ds(base, _RPW)], idxv)
        bufs, sems = (b0, b1, b2, b3), (s0, s1, s2, s3)
        gops = [None] * _DEPTH
        for c in range(_NCH + _DEPTH - 1):
            if c < _NCH:
                p = c % _DEPTH
                gops[p] = pltpu.async_copy(
                    tab_h.at[idxv.at[pl.ds(c * _CH, _CH)]], bufs[p], sems[p])
            if c >= _DEPTH - 1:
                cc = c - _DEPTH + 1
                q = cc % _DEPTH
                gops[q].wait()
                pltpu.sync_copy(bufs[q], gt_h.at[pl.ds(base + cc * _CH, _CH)])

    return gk(tab, idx)


def _rep_rows(x, tq):
    c = x.shape[-1]
    return jnp.broadcast_to(x[:, None, :], (tq, K_NEI, c)).reshape(tq * K_NEI, c)


def _group_sum(x, tq):
    return jnp.sum(x.reshape(tq, K_NEI, x.shape[-1]), axis=1)


_TQ1 = 256
_TR1 = _TQ1 * K_NEI
_NT1 = (B * N) // _TQ1


def _rstat_body(gt_ref, pos_ref, sr_ref, srr_ref):
    t = pl.program_id(0)
    prep = _rep_rows(pos_ref[...], _TQ1)
    r = prep - gt_ref[...][:, 2 * DIM:2 * DIM + 3]
    sr = jnp.sum(r, axis=0, keepdims=True)
    srr = lax.dot_general(r, r, (((0,), (0,)), ((), ())),
                          preferred_element_type=_F32, precision=_PREC)

    @pl.when(t == 0)
    def _():
        sr_ref[...] = jnp.zeros_like(sr_ref)
        srr_ref[...] = jnp.zeros_like(srr_ref)

    sr_ref[...] += sr
    srr_ref[...] += srr


def _run_rstat(gt, pos2d):
    return pl.pallas_call(
        _rstat_body,
        grid=(_NT1,),
        in_specs=[
            pl.BlockSpec((_TR1, _TW), lambda t: (t, 0)),
            pl.BlockSpec((_TQ1, 3), lambda t: (t, 0)),
        ],
        out_specs=[
            pl.BlockSpec((1, 3), lambda t: (0, 0)),
            pl.BlockSpec((3, 3), lambda t: (0, 0)),
        ],
        out_shape=[
            jax.ShapeDtypeStruct((1, 3), _F32),
            jax.ShapeDtypeStruct((3, 3), _F32),
        ],
        compiler_params=pltpu.CompilerParams(
            dimension_semantics=("arbitrary",)),
    )(gt, pos2d)


_TQ2 = 128
_TR2 = _TQ2 * K_NEI
_NT2 = (B * N) // _TQ2


def _ustat_body(gt_ref, q_ref, pos_ref, w1T_ref, w2T_ref, b2_ref,
                a1_ref, c1_ref, su_ref, suu_ref):
    t = pl.program_id(0)
    gt = gt_ref[...]
    prep = _rep_rows(pos_ref[...], _TQ2)
    r = prep - gt[:, 2 * DIM:2 * DIM + 3]
    h = jnp.dot(r, w1T_ref[...], preferred_element_type=_F32, precision=_PREC)
    h = jnp.maximum(h * a1_ref[...] + c1_ref[...], 0.0)
    rpe = jnp.dot(h, w2T_ref[...], preferred_element_type=_F32, precision=_PREC) + b2_ref[...]
    qrep = _rep_rows(q_ref[...], _TQ2)
    u = qrep - gt[:, 0:DIM] + rpe
    su = jnp.sum(u, axis=0, keepdims=True)
    suu = lax.dot_general(u, u, (((0,), (0,)), ((), ())),
                          preferred_element_type=_F32, precision=_PREC)

    @pl.when(t == 0)
    def _():
        su_ref[...] = jnp.zeros_like(su_ref)
        suu_ref[...] = jnp.zeros_like(suu_ref)

    su_ref[...] += su
    suu_ref[...] += suu


def _run_ustat(gt, q2d, pos2d, w1T, w2T, b2, a1, c1):
    return pl.pallas_call(
        _ustat_body,
        grid=(_NT2,),
        in_specs=[
            pl.BlockSpec((_TR2, _TW), lambda t: (t, 0)),
            pl.BlockSpec((_TQ2, DIM), lambda t: (t, 0)),
            pl.BlockSpec((_TQ2, 3), lambda t: (t, 0)),
            pl.BlockSpec((3, POS_HID), lambda t: (0, 0)),
            pl.BlockSpec((POS_HID, DIM), lambda t: (0, 0)),
            pl.BlockSpec((1, DIM), lambda t: (0, 0)),
            pl.BlockSpec((1, POS_HID), lambda t: (0, 0)),
            pl.BlockSpec((1, POS_HID), lambda t: (0, 0)),
        ],
        out_specs=[
            pl.BlockSpec((1, DIM), lambda t: (0, 0)),
            pl.BlockSpec((DIM, DIM), lambda t: (0, 0)),
        ],
        out_shape=[
            jax.ShapeDtypeStruct((1, DIM), _F32),
            jax.ShapeDtypeStruct((DIM, DIM), _F32),
        ],
        compiler_params=pltpu.CompilerParams(
            dimension_semantics=("arbitrary",)),
    )(gt, q2d, pos2d, w1T, w2T, b2, a1, c1)


def _fin_body(s_ref, ss_ref, w1T_ref, b1_ref, g_ref, beta_ref, scale_ref, shift_ref):
    m = s_ref[...] * (1.0 / M)
    cov = ss_ref[...] * (1.0 / M) - lax.dot_general(
        m, m, (((0,), (0,)), ((), ())), preferred_element_type=_F32, precision=_PREC)
    tmp = jnp.dot(cov, w1T_ref[...], preferred_element_type=_F32, precision=_PREC)
    var = jnp.sum(tmp * w1T_ref[...], axis=0, keepdims=True)
    mean = jnp.dot(m, w1T_ref[...], preferred_element_type=_F32,
                   precision=_PREC) + b1_ref[...]
    scale = g_ref[...] * lax.rsqrt(var + EPS)
    scale_ref[...] = scale
    shift_ref[...] = beta_ref[...] + (b1_ref[...] - mean) * scale


def _run_fin(s, ss, w1T, b1, g, beta):
    din, dout = w1T.shape
    full = lambda shape: pl.BlockSpec(shape, lambda: tuple(0 for _ in shape))
    return pl.pallas_call(
        _fin_body,
        in_specs=[full((1, din)), full((din, din)), full((din, dout)),
                  full((1, dout)), full((1, dout)), full((1, dout))],
        out_specs=[full((1, dout)), full((1, dout))],
        out_shape=[jax.ShapeDtypeStruct((1, dout), _F32),
                   jax.ShapeDtypeStruct((1, dout), _F32)],
    )(s, ss, w1T, b1, g, beta)


_TQ3 = 128
_TR3 = _TQ3 * K_NEI
_NT3 = (B * N) // _TQ3
_INV_SQRT_DIM = 0.125


def _fwd_body(gt_ref, q_ref, pos_ref, ori_ref,
              w1T_ref, w2T_ref, b2_ref, a1_ref, c1_ref,
              aw1T_ref, aw2T_ref, ab2_ref, a2_ref, c2_ref, woT_ref, out_ref):
    gt = gt_ref[...]
    prep = _rep_rows(pos_ref[...], _TQ3)
    r = prep - gt[:, 2 * DIM:2 * DIM + 3]
    h = jnp.dot(r, w1T_ref[...], preferred_element_type=_F32, precision=_PREC)
    h = jnp.maximum(h * a1_ref[...] + c1_ref[...], 0.0)
    rpe = jnp.dot(h, w2T_ref[...], preferred_element_type=_F32, precision=_PREC) + b2_ref[...]
    qrep = _rep_rows(q_ref[...], _TQ3)
    u = qrep - gt[:, 0:DIM] + rpe
    a = jnp.dot(u, aw1T_ref[...], preferred_element_type=_F32, precision=_PREC)
    a = jnp.maximum(a * a2_ref[...] + c2_ref[...], 0.0)
    attn = (jnp.dot(a, aw2T_ref[...], preferred_element_type=_F32, precision=_PREC)
            + ab2_ref[...]) * _INV_SQRT_DIM
    e = jnp.exp(attn)
    denom = _group_sum(e, _TQ3)
    w = gt[:, DIM:2 * DIM] + rpe
    num = _group_sum(e * w, _TQ3)
    res = num / denom
    out_ref[...] = jnp.dot(res, woT_ref[...], preferred_element_type=_F32,
                           precision=_PREC) + ori_ref[...]


def _run_fwd(gt, q2d, pos2d, ori2d, w1T, w2T, b2, a1, c1,
             aw1T, aw2T, ab2, a2, c2, woT):
    return pl.pallas_call(
        _fwd_body,
        grid=(_NT3,),
        in_specs=[
            pl.BlockSpec((_TR3, _TW), lambda t: (t, 0)),
            pl.BlockSpec((_TQ3, DIM), lambda t: (t, 0)),
            pl.BlockSpec((_TQ3, 3), lambda t: (t, 0)),
            pl.BlockSpec((_TQ3, IN_DIM), lambda t: (t, 0)),
            pl.BlockSpec((3, POS_HID), lambda t: (0, 0)),
            pl.BlockSpec((POS_HID, DIM), lambda t: (0, 0)),
            pl.BlockSpec((1, DIM), lambda t: (0, 0)),
            pl.BlockSpec((1, POS_HID), lambda t: (0, 0)),
            pl.BlockSpec((1, POS_HID), lambda t: (0, 0)),
            pl.BlockSpec((DIM, ATTN_HID), lambda t: (0, 0)),
            pl.BlockSpec((ATTN_HID, DIM), lambda t: (0, 0)),
            pl.BlockSpec((1, DIM), lambda t: (0, 0)),
            pl.BlockSpec((1, ATTN_HID), lambda t: (0, 0)),
            pl.BlockSpec((1, ATTN_HID), lambda t: (0, 0)),
            pl.BlockSpec((DIM, IN_DIM), lambda t: (0, 0)),
        ],
        out_specs=pl.BlockSpec((_TQ3, IN_DIM), lambda t: (t, 0)),
        out_shape=jax.ShapeDtypeStruct((B * N, IN_DIM), _F32),
        compiler_params=pltpu.CompilerParams(
            dimension_semantics=("parallel",)),
    )(gt, q2d, pos2d, ori2d, w1T, w2T, b2, a1, c1,
      aw1T, aw2T, ab2, a2, c2, woT)


def kernel(ori_x, pos, W_in, W_qkv, W_out, pm_w1, pm_b1, pm_g, pm_beta, pm_w2, pm_b2,
           am_w1, am_b1, am_g, am_beta, am_w2, am_b2):
    post = jnp.swapaxes(pos, 1, 2)
    q, kvp, gidx = _run_knn(ori_x, pos, post, W_in.T, W_qkv.T)

    pos2d = pos.reshape(B * N, 3)
    gt = _sc_gather(kvp.reshape(B * N, _TW), gidx.reshape(M))

    sr, srr = _run_rstat(gt, pos2d)
    scale1, shift1 = _run_fin(sr, srr, pm_w1.T, pm_b1[None, :], pm_g[None, :],
                              pm_beta[None, :])

    q2d = q.reshape(B * N, DIM)
    su, suu = _run_ustat(gt, q2d, pos2d, pm_w1.T, pm_w2.T, pm_b2[None, :],
                         scale1, shift1)
    scale2, shift2 = _run_fin(su, suu, am_w1.T, am_b1[None, :], am_g[None, :],
                              am_beta[None, :])

    out2d = _run_fwd(gt, q2d, pos2d, ori_x.reshape(B * N, IN_DIM),
                     pm_w1.T, pm_w2.T, pm_b2[None, :], scale1, shift1,
                     am_w1.T, am_w2.T, am_b2[None, :], scale2, shift2, W_out.T)
    return out2d.reshape(B, N, IN_DIM)
